# baseline (speedup 1.0000x reference)
import functools

import jax
import jax.numpy as jnp
from jax import lax
from jax.experimental import pallas as pl
from jax.experimental.pallas import tpu as pltpu
from jax.experimental.pallas import tpu_sc as plsc

N_NODES = 10000
N_EDGES = 80000
EDGE_DIM = 16
EMBED = 32
EPS = 1e-5

NUM_CORES = 2
NUM_SUB = 16
NW = NUM_CORES * NUM_SUB
E_PAD = 81920
PER_W = E_PAD // NW
BATCH = 128
NB = PER_W // BATCH
N_PAD = 10240
ROWS_PER_SUB = N_PAD // NUM_SUB
T_MSG = 4096
T_NODE = 2000


def _sc_mesh():
    return plsc.VectorSubcoreMesh(
        core_axis_name="c", subcore_axis_name="s",
        num_cores=NUM_CORES, num_subcores=NUM_SUB)


NSLOT = 4


@functools.cache
def _gather_fn(in_ch):
    def body(x_hbm, idx_hbm, out_hbm, idx_v, rows_v, *sems):
        gsems, wsems = sems[:NSLOT], sems[NSLOT:]
        c = lax.axis_index("c")
        s = lax.axis_index("s")
        w = c * NUM_SUB + s
        pltpu.sync_copy(idx_hbm.at[w], idx_v)

        def g_start(j):
            return pltpu.async_copy(
                x_hbm.at[idx_v.at[j]], rows_v.at[j % NSLOT], gsems[j % NSLOT])

        def w_start(j):
            return pltpu.async_copy(
                rows_v.at[j % NSLOT],
                out_hbm.at[pl.ds(w * PER_W + j * BATCH, BATCH)],
                wsems[j % NSLOT])

        gd = [None] * NB
        wd = [None] * NB
        for j in range(NB):
            if j >= NSLOT:
                wd[j - NSLOT].wait()
            gd[j] = g_start(j)
            k = j - 2
            if k >= 0:
                gd[k].wait()
                wd[k] = w_start(k)
        for k in range(NB - 2, NB):
            gd[k].wait()
            wd[k] = w_start(k)
        for k in range(NB - NSLOT, NB):
            wd[k].wait()

    return pl.kernel(
        body,
        out_type=jax.ShapeDtypeStruct((E_PAD, in_ch), jnp.float32),
        mesh=_sc_mesh(),
        compiler_params=pltpu.CompilerParams(use_tc_tiling_on_sc=False),
        scratch_types=[
            pltpu.VMEM((NB, BATCH), jnp.int32),
            pltpu.VMEM((NSLOT, BATCH, in_ch), jnp.float32),
        ] + [pltpu.SemaphoreType.DMA] * (2 * NSLOT))


@functools.cache
def _scatter_fn():
    def body(msg_hbm, idx_hbm, zeros_hbm, out_hbm, shared, idx_v, msg_v, ssem):
        c = lax.axis_index("c")
        s = lax.axis_index("s")
        w = c * NUM_SUB + s
        pltpu.sync_copy(zeros_hbm, shared.at[pl.ds(s * ROWS_PER_SUB, ROWS_PER_SUB)])
        plsc.subcore_barrier()
        pltpu.sync_copy(msg_hbm.at[pl.ds(w * PER_W, PER_W)], msg_v)
        pltpu.sync_copy(idx_hbm.at[w], idx_v)

        def step(j, carry):
            pltpu.async_copy(msg_v.at[pl.ds(j * BATCH, BATCH)],
                             shared.at[idx_v.at[j]], ssem, add=True).wait()
            return carry

        lax.fori_loop(0, NB, step, 0)
        plsc.subcore_barrier()
        pltpu.sync_copy(shared.at[pl.ds(s * ROWS_PER_SUB, ROWS_PER_SUB)],
                        out_hbm.at[c, pl.ds(s * ROWS_PER_SUB, ROWS_PER_SUB)])

    return pl.kernel(
        body,
        out_type=jax.ShapeDtypeStruct((NUM_CORES, N_PAD, EMBED), jnp.float32),
        mesh=_sc_mesh(),
        compiler_params=pltpu.CompilerParams(use_tc_tiling_on_sc=False),
        scratch_types=[
            pltpu.VMEM_SHARED((N_PAD, EMBED), jnp.float32),
            pltpu.VMEM((NB, BATCH), jnp.int32),
            pltpu.VMEM((PER_W, EMBED), jnp.float32),
            pltpu.SemaphoreType.DMA,
        ])


def _bn_derive(m_acc, s_acc, w1t, b1, g, be):
    mE = m_acc / N_EDGES
    SE = s_acc / N_EDGES
    t = jnp.dot(mE, w1t, preferred_element_type=jnp.float32)
    mean = t + b1
    bq = jnp.dot(SE, w1t, preferred_element_type=jnp.float32)
    q = jnp.sum(w1t * bq, axis=0, keepdims=True)
    var = (q + 2.0 * b1 * t + b1 * b1) - mean * mean
    scale = g * lax.rsqrt(var + EPS)
    shift = be + scale * (b1 - mean)
    return scale, shift


@functools.cache
def _msg0_fn():
    in_ch = 64
    NT = E_PAD // T_MSG
    grid = (2 * NT,)

    def body(ea_ref, xs_ref, w1tr_ref, w2q_ref, b2r_ref, repmat_ref,
             w1t0_ref, b10_ref, g0_ref, be0_ref,
             w1t1_ref, b11_ref, g1_ref, be1_ref,
             msg_ref, ss1_ref, m_acc, s_acc, screp, shrep):
        i = pl.program_id(0)

        @pl.when(i == 0)
        def _():
            m_acc[...] = jnp.zeros_like(m_acc)
            s_acc[...] = jnp.zeros_like(s_acc)

        @pl.when(i < NT)
        def _():
            ea_t = ea_ref[...]
            m_acc[...] += jnp.sum(ea_t, axis=0, keepdims=True)
            s_acc[...] += lax.dot_general(ea_t, ea_t, (((0,), (0,)), ((), ())),
                                          preferred_element_type=jnp.float32)

        @pl.when(i == NT - 1)
        def _():
            sc0, sh0 = _bn_derive(m_acc[...], s_acc[...], w1t0_ref[...],
                                  b10_ref[...], g0_ref[...], be0_ref[...])
            screp[...] = jnp.dot(sc0, repmat_ref[...],
                                 preferred_element_type=jnp.float32)
            shrep[...] = jnp.dot(sh0, repmat_ref[...],
                                 preferred_element_type=jnp.float32)
            sc1, sh1 = _bn_derive(m_acc[...], s_acc[...], w1t1_ref[...],
                                  b11_ref[...], g1_ref[...], be1_ref[...])
            ss1_ref[...] = jnp.concatenate([sc1, sh1], axis=0)

        @pl.when(i >= NT)
        def _():
            h0 = jnp.dot(ea_ref[...], w1tr_ref[...],
                         preferred_element_type=jnp.float32)
            h_rep = jnp.maximum(h0 * screp[...] + shrep[...], 0.0)
            xs = xs_ref[...]
            t2 = jnp.dot(xs.astype(jnp.bfloat16), w2q_ref[...],
                         preferred_element_type=jnp.float32)
            p = h_rep * t2
            width = EMBED * EMBED
            while width > EMBED:
                width //= 2
                p = p[:, :width] + p[:, width:2 * width]
            msg_ref[...] = p + jnp.dot(xs, b2r_ref[...],
                                       preferred_element_type=jnp.float32)

    small = lambda shape: pl.BlockSpec(shape, lambda i: (0,) * len(shape))
    edge = lambda width: pl.BlockSpec((T_MSG, width),
                                      lambda i: (jnp.maximum(i - NT, 0), 0))
    return pl.pallas_call(
        body,
        grid=grid,
        in_specs=[pl.BlockSpec((T_MSG, EDGE_DIM), lambda i: (i % NT, 0)),
                  edge(in_ch),
                  small((EDGE_DIM, EMBED * EMBED)),
                  small((in_ch, EMBED * EMBED)),
                  small((in_ch, EMBED)),
                  small((EMBED, EMBED * EMBED)),
                  small((EDGE_DIM, EMBED)), small((1, EMBED)),
                  small((1, EMBED)), small((1, EMBED)),
                  small((EDGE_DIM, EMBED)), small((1, EMBED)),
                  small((1, EMBED)), small((1, EMBED))],
        out_specs=[edge(EMBED), small((2, EMBED))],
        out_shape=[jax.ShapeDtypeStruct((E_PAD, EMBED), jnp.float32),
                   jax.ShapeDtypeStruct((2, EMBED), jnp.float32)],
        scratch_shapes=[pltpu.VMEM((1, EDGE_DIM), jnp.float32),
                        pltpu.VMEM((EDGE_DIM, EDGE_DIM), jnp.float32),
                        pltpu.VMEM((1, EMBED * EMBED), jnp.float32),
                        pltpu.VMEM((1, EMBED * EMBED), jnp.float32)],
        compiler_params=pltpu.CompilerParams(
            dimension_semantics=("arbitrary",)),
    )


@functools.cache
def _msg_fn(in_ch):
    grid = (E_PAD // T_MSG,)

    def body(ea_ref, xs_ref, w1ts_ref, w2q_ref, b2r_ref, sh_ref, msg_ref):
        h0 = jnp.dot(ea_ref[...], w1ts_ref[...],
                     preferred_element_type=jnp.float32)
        h_rep = jnp.maximum(h0 + sh_ref[...], 0.0)
        xs = xs_ref[...]
        t2 = jnp.dot(xs.astype(jnp.bfloat16), w2q_ref[...],
                     preferred_element_type=jnp.float32)
        p = h_rep * t2
        width = EMBED * EMBED
        while width > EMBED:
            width //= 2
            p = p[:, :width] + p[:, width:2 * width]
        msg_ref[...] = p + jnp.dot(xs, b2r_ref[...],
                                   preferred_element_type=jnp.float32)

    small = lambda shape: pl.BlockSpec(shape, lambda i: (0, 0))
    return pl.pallas_call(
        body,
        grid=grid,
        in_specs=[pl.BlockSpec((T_MSG, EDGE_DIM), lambda i: (i, 0)),
                  pl.BlockSpec((T_MSG, in_ch), lambda i: (i, 0)),
                  small((EDGE_DIM, EMBED * EMBED)),
                  small((in_ch, EMBED * EMBED)),
                  small((in_ch, EMBED)),
                  small((1, EMBED * EMBED))],
        out_specs=pl.BlockSpec((T_MSG, EMBED), lambda i: (i, 0)),
        out_shape=jax.ShapeDtypeStruct((E_PAD, EMBED), jnp.float32),
        compiler_params=pltpu.CompilerParams(
            dimension_semantics=("arbitrary",)),
    )


@functools.cache
def _node_fn(in_ch):
    grid = (N_NODES // T_NODE,)

    def body(agg_ref, x_ref, root_ref, bias_ref, out_ref):
        a = agg_ref[0] + agg_ref[1]
        r = jnp.dot(x_ref[...], root_ref[...], preferred_element_type=jnp.float32)
        out_ref[...] = jnp.maximum(a + r + bias_ref[...], 0.0)

    small = lambda shape: pl.BlockSpec(shape, lambda *_: (0,) * len(shape))
    return pl.pallas_call(
        body,
        grid=grid,
        in_specs=[pl.BlockSpec((NUM_CORES, T_NODE, EMBED), lambda i: (0, i, 0)),
                  pl.BlockSpec((T_NODE, in_ch), lambda i: (i, 0)),
                  small((in_ch, EMBED)),
                  small((1, EMBED))],
        out_specs=pl.BlockSpec((T_NODE, EMBED), lambda i: (i, 0)),
        out_shape=jax.ShapeDtypeStruct((N_NODES, EMBED), jnp.float32),
        compiler_params=pltpu.CompilerParams(
            dimension_semantics=("arbitrary",)),
    )


def kernel(node_attr, edge_index, edge_attr,
           w1_0, b1_0, gamma_0, beta_0, w2_0, b2_0, root_0, bias_0,
           w1_1, b1_1, gamma_1, beta_1, w2_1, b2_1, root_1, bias_1):
    src = edge_index[0].astype(jnp.int32)
    dst = edge_index[1].astype(jnp.int32)
    src2d = jnp.pad(src, (0, E_PAD - N_EDGES)).reshape(NW, NB, BATCH)
    dst2d = jnp.pad(dst, (0, E_PAD - N_EDGES),
                    constant_values=N_NODES).reshape(NW, NB, BATCH)
    ea_pad = jnp.pad(edge_attr, ((0, E_PAD - N_EDGES), (0, 0)))
    zeros_blk = jnp.zeros((ROWS_PER_SUB, EMBED), jnp.float32)

    r2 = lambda v: v.reshape(1, EMBED)

    def prep(w2, b2, in_ch):
        w2q = (w2.reshape(in_ch, EMBED, EMBED)
               .transpose(0, 2, 1).reshape(in_ch, EMBED * EMBED))
        return w2q.astype(jnp.bfloat16), b2.reshape(in_ch, EMBED)

    w1t0, w1t1 = w1_0.T, w1_1.T
    w2q0, b2r0 = prep(w2_0, b2_0, 64)
    w2q1, b2r1 = prep(w2_1, b2_1, EMBED)

    rep = lambda v: jnp.repeat(v, EMBED, axis=1)
    repmat = jnp.repeat(jnp.eye(EMBED, dtype=jnp.float32), EMBED, axis=1)

    xs0 = _gather_fn(64)(node_attr, src2d).reshape(E_PAD, 64)
    msg0, ss1 = _msg0_fn()(ea_pad, xs0, rep(w1t0), w2q0, b2r0, repmat,
                           w1t0, r2(b1_0), r2(gamma_0), r2(beta_0),
                           w1t1, r2(b1_1), r2(gamma_1), r2(beta_1))
    agg0 = _scatter_fn()(msg0, dst2d, zeros_blk)
    x1 = _node_fn(64)(agg0, node_attr, root_0, r2(bias_0))

    w1ts1 = rep(w1t1) * rep(ss1[0:1])
    xs1 = _gather_fn(EMBED)(x1, src2d).reshape(E_PAD, EMBED)
    msg1 = _msg_fn(EMBED)(ea_pad, xs1, w1ts1, w2q1, b2r1, rep(ss1[1:2]))
    agg1 = _scatter_fn()(msg1, dst2d, zeros_blk)
    return _node_fn(EMBED)(agg1, x1, root_1, r2(bias_1))


# speedup vs baseline: 1.0116x; 1.0116x over previous
import functools

import jax
import jax.numpy as jnp
from jax import lax
from jax.experimental import pallas as pl
from jax.experimental.pallas import tpu as pltpu
from jax.experimental.pallas import tpu_sc as plsc

N_NODES = 10000
N_EDGES = 80000
EDGE_DIM = 16
EMBED = 32
EPS = 1e-5

NUM_CORES = 2
NUM_SUB = 16
NW = NUM_CORES * NUM_SUB
E_PAD = 81920
PER_W = E_PAD // NW
BATCH = 128
NB = PER_W // BATCH
N_PAD = 10240
ROWS_PER_SUB = N_PAD // NUM_SUB
T_MSG = 4096
T_NODE = 2000


def _sc_mesh():
    return plsc.VectorSubcoreMesh(
        core_axis_name="c", subcore_axis_name="s",
        num_cores=NUM_CORES, num_subcores=NUM_SUB)


NSLOT = 4


@functools.cache
def _gather_fn(in_ch):
    def body(x_hbm, idx_hbm, out_hbm, idx_v, rows_v, *sems):
        gsems, wsems = sems[:NSLOT], sems[NSLOT:]
        c = lax.axis_index("c")
        s = lax.axis_index("s")
        w = c * NUM_SUB + s
        pltpu.sync_copy(idx_hbm.at[w], idx_v)

        def g_start(j):
            return pltpu.async_copy(
                x_hbm.at[idx_v.at[j]], rows_v.at[j % NSLOT], gsems[j % NSLOT])

        def w_start(j):
            return pltpu.async_copy(
                rows_v.at[j % NSLOT],
                out_hbm.at[pl.ds(w * PER_W + j * BATCH, BATCH)],
                wsems[j % NSLOT])

        DEPTH = 3
        gd = [None] * NB
        wd = [None] * NB
        for j in range(NB):
            if j >= NSLOT:
                wd[j - NSLOT].wait()
            gd[j] = g_start(j)
            k = j - DEPTH
            if k >= 0:
                gd[k].wait()
                wd[k] = w_start(k)
        for k in range(NB - DEPTH, NB):
            gd[k].wait()
            wd[k] = w_start(k)
        for k in range(NB - NSLOT, NB):
            wd[k].wait()

    return pl.kernel(
        body,
        out_type=jax.ShapeDtypeStruct((E_PAD, in_ch), jnp.bfloat16),
        mesh=_sc_mesh(),
        compiler_params=pltpu.CompilerParams(use_tc_tiling_on_sc=False),
        scratch_types=[
            pltpu.VMEM((NB, BATCH), jnp.int32),
            pltpu.VMEM((NSLOT, BATCH, in_ch), jnp.bfloat16),
        ] + [pltpu.SemaphoreType.DMA] * (2 * NSLOT))


@functools.cache
def _scatter_fn():
    def body(msg_hbm, idx_hbm, zeros_hbm, out_hbm, shared, idx_v, msg_v, ssem):
        c = lax.axis_index("c")
        s = lax.axis_index("s")
        w = c * NUM_SUB + s
        pltpu.sync_copy(zeros_hbm, shared.at[pl.ds(s * ROWS_PER_SUB, ROWS_PER_SUB)])
        plsc.subcore_barrier()
        pltpu.sync_copy(msg_hbm.at[pl.ds(w * PER_W, PER_W)], msg_v)
        pltpu.sync_copy(idx_hbm.at[w], idx_v)

        def step(j, carry):
            pltpu.async_copy(msg_v.at[pl.ds(j * BATCH, BATCH)],
                             shared.at[idx_v.at[j]], ssem, add=True).wait()
            return carry

        lax.fori_loop(0, NB, step, 0)
        plsc.subcore_barrier()
        pltpu.sync_copy(shared.at[pl.ds(s * ROWS_PER_SUB, ROWS_PER_SUB)],
                        out_hbm.at[c, pl.ds(s * ROWS_PER_SUB, ROWS_PER_SUB)])

    return pl.kernel(
        body,
        out_type=jax.ShapeDtypeStruct((NUM_CORES, N_PAD, EMBED), jnp.float32),
        mesh=_sc_mesh(),
        compiler_params=pltpu.CompilerParams(use_tc_tiling_on_sc=False),
        scratch_types=[
            pltpu.VMEM_SHARED((N_PAD, EMBED), jnp.float32),
            pltpu.VMEM((NB, BATCH), jnp.int32),
            pltpu.VMEM((PER_W, EMBED), jnp.float32),
            pltpu.SemaphoreType.DMA,
        ])


def _bn_derive(m_acc, s_acc, w1t, b1, g, be):
    mE = m_acc / N_EDGES
    SE = s_acc / N_EDGES
    t = jnp.dot(mE, w1t, preferred_element_type=jnp.float32)
    mean = t + b1
    bq = jnp.dot(SE, w1t, preferred_element_type=jnp.float32)
    q = jnp.sum(w1t * bq, axis=0, keepdims=True)
    var = (q + 2.0 * b1 * t + b1 * b1) - mean * mean
    scale = g * lax.rsqrt(var + EPS)
    shift = be + scale * (b1 - mean)
    return scale, shift


@functools.cache
def _msg0_fn():
    in_ch = 64
    NT = E_PAD // T_MSG
    grid = (2 * NT,)

    def body(ea_ref, xs_ref, w1tr_ref, w2q_ref, b2r_ref, repmat_ref,
             w1t0_ref, b10_ref, g0_ref, be0_ref,
             w1t1_ref, b11_ref, g1_ref, be1_ref,
             msg_ref, ss1_ref, m_acc, s_acc, screp, shrep):
        i = pl.program_id(0)

        @pl.when(i == 0)
        def _():
            m_acc[...] = jnp.zeros_like(m_acc)
            s_acc[...] = jnp.zeros_like(s_acc)

        @pl.when(i < NT)
        def _():
            ea_t = ea_ref[...]
            m_acc[...] += jnp.sum(ea_t, axis=0, keepdims=True)
            s_acc[...] += lax.dot_general(ea_t, ea_t, (((0,), (0,)), ((), ())),
                                          preferred_element_type=jnp.float32)

        @pl.when(i == NT - 1)
        def _():
            sc0, sh0 = _bn_derive(m_acc[...], s_acc[...], w1t0_ref[...],
                                  b10_ref[...], g0_ref[...], be0_ref[...])
            screp[...] = jnp.dot(sc0, repmat_ref[...],
                                 preferred_element_type=jnp.float32)
            shrep[...] = jnp.dot(sh0, repmat_ref[...],
                                 preferred_element_type=jnp.float32)
            sc1, sh1 = _bn_derive(m_acc[...], s_acc[...], w1t1_ref[...],
                                  b11_ref[...], g1_ref[...], be1_ref[...])
            ss1_ref[...] = jnp.concatenate([sc1, sh1], axis=0)

        @pl.when(i >= NT)
        def _():
            h0 = jnp.dot(ea_ref[...], w1tr_ref[...],
                         preferred_element_type=jnp.float32)
            h_rep = jnp.maximum(h0 * screp[...] + shrep[...], 0.0)
            xs = xs_ref[...]
            t2 = jnp.dot(xs, w2q_ref[...],
                         preferred_element_type=jnp.float32)
            p = h_rep * t2
            width = EMBED * EMBED
            while width > EMBED:
                width //= 2
                p = p[:, :width] + p[:, width:2 * width]
            msg_ref[...] = p + jnp.dot(xs, b2r_ref[...],
                                       preferred_element_type=jnp.float32)

    small = lambda shape: pl.BlockSpec(shape, lambda i: (0,) * len(shape))
    edge = lambda width: pl.BlockSpec((T_MSG, width),
                                      lambda i: (jnp.maximum(i - NT, 0), 0))
    return pl.pallas_call(
        body,
        grid=grid,
        in_specs=[pl.BlockSpec((T_MSG, EDGE_DIM), lambda i: (i % NT, 0)),
                  edge(in_ch),
                  small((EDGE_DIM, EMBED * EMBED)),
                  small((in_ch, EMBED * EMBED)),
                  small((in_ch, EMBED)),
                  small((EMBED, EMBED * EMBED)),
                  small((EDGE_DIM, EMBED)), small((1, EMBED)),
                  small((1, EMBED)), small((1, EMBED)),
                  small((EDGE_DIM, EMBED)), small((1, EMBED)),
                  small((1, EMBED)), small((1, EMBED))],
        out_specs=[edge(EMBED), small((2, EMBED))],
        out_shape=[jax.ShapeDtypeStruct((E_PAD, EMBED), jnp.float32),
                   jax.ShapeDtypeStruct((2, EMBED), jnp.float32)],
        scratch_shapes=[pltpu.VMEM((1, EDGE_DIM), jnp.float32),
                        pltpu.VMEM((EDGE_DIM, EDGE_DIM), jnp.float32),
                        pltpu.VMEM((1, EMBED * EMBED), jnp.float32),
                        pltpu.VMEM((1, EMBED * EMBED), jnp.float32)],
        compiler_params=pltpu.CompilerParams(
            dimension_semantics=("arbitrary",)),
    )


@functools.cache
def _msg_fn(in_ch):
    grid = (E_PAD // T_MSG,)

    def body(ea_ref, xs_ref, w1ts_ref, w2q_ref, b2r_ref, sh_ref, msg_ref):
        h0 = jnp.dot(ea_ref[...], w1ts_ref[...],
                     preferred_element_type=jnp.float32)
        h_rep = jnp.maximum(h0 + sh_ref[...], 0.0)
        xs = xs_ref[...]
        t2 = jnp.dot(xs, w2q_ref[...],
                     preferred_element_type=jnp.float32)
        p = h_rep * t2
        width = EMBED * EMBED
        while width > EMBED:
            width //= 2
            p = p[:, :width] + p[:, width:2 * width]
        msg_ref[...] = p + jnp.dot(xs, b2r_ref[...],
                                   preferred_element_type=jnp.float32)

    small = lambda shape: pl.BlockSpec(shape, lambda i: (0, 0))
    return pl.pallas_call(
        body,
        grid=grid,
        in_specs=[pl.BlockSpec((T_MSG, EDGE_DIM), lambda i: (i, 0)),
                  pl.BlockSpec((T_MSG, in_ch), lambda i: (i, 0)),
                  small((EDGE_DIM, EMBED * EMBED)),
                  small((in_ch, EMBED * EMBED)),
                  small((in_ch, EMBED)),
                  small((1, EMBED * EMBED))],
        out_specs=pl.BlockSpec((T_MSG, EMBED), lambda i: (i, 0)),
        out_shape=jax.ShapeDtypeStruct((E_PAD, EMBED), jnp.float32),
        compiler_params=pltpu.CompilerParams(
            dimension_semantics=("arbitrary",)),
    )


@functools.cache
def _node_fn(in_ch):
    grid = (N_NODES // T_NODE,)

    def body(agg_ref, x_ref, root_ref, bias_ref, out_ref, outb_ref):
        a = agg_ref[0] + agg_ref[1]
        r = jnp.dot(x_ref[...], root_ref[...], preferred_element_type=jnp.float32)
        res = jnp.maximum(a + r + bias_ref[...], 0.0)
        out_ref[...] = res
        outb_ref[...] = res.astype(jnp.bfloat16)

    small = lambda shape: pl.BlockSpec(shape, lambda *_: (0,) * len(shape))
    return pl.pallas_call(
        body,
        grid=grid,
        in_specs=[pl.BlockSpec((NUM_CORES, T_NODE, EMBED), lambda i: (0, i, 0)),
                  pl.BlockSpec((T_NODE, in_ch), lambda i: (i, 0)),
                  small((in_ch, EMBED)),
                  small((1, EMBED))],
        out_specs=[pl.BlockSpec((T_NODE, EMBED), lambda i: (i, 0)),
                   pl.BlockSpec((T_NODE, EMBED), lambda i: (i, 0))],
        out_shape=[jax.ShapeDtypeStruct((N_NODES, EMBED), jnp.float32),
                   jax.ShapeDtypeStruct((N_NODES, EMBED), jnp.bfloat16)],
        compiler_params=pltpu.CompilerParams(
            dimension_semantics=("arbitrary",)),
    )


def kernel(node_attr, edge_index, edge_attr,
           w1_0, b1_0, gamma_0, beta_0, w2_0, b2_0, root_0, bias_0,
           w1_1, b1_1, gamma_1, beta_1, w2_1, b2_1, root_1, bias_1):
    src = edge_index[0].astype(jnp.int32)
    dst = edge_index[1].astype(jnp.int32)
    src2d = jnp.pad(src, (0, E_PAD - N_EDGES)).reshape(NW, NB, BATCH)
    dst2d = jnp.pad(dst, (0, E_PAD - N_EDGES),
                    constant_values=N_NODES).reshape(NW, NB, BATCH)
    ea_pad = jnp.pad(edge_attr, ((0, E_PAD - N_EDGES), (0, 0)))
    zeros_blk = jnp.zeros((ROWS_PER_SUB, EMBED), jnp.float32)

    r2 = lambda v: v.reshape(1, EMBED)

    def prep(w2, b2, in_ch):
        w2q = (w2.reshape(in_ch, EMBED, EMBED)
               .transpose(0, 2, 1).reshape(in_ch, EMBED * EMBED))
        return (w2q.astype(jnp.bfloat16),
                b2.reshape(in_ch, EMBED).astype(jnp.bfloat16))

    w1t0, w1t1 = w1_0.T, w1_1.T
    w2q0, b2r0 = prep(w2_0, b2_0, 64)
    w2q1, b2r1 = prep(w2_1, b2_1, EMBED)

    rep = lambda v: jnp.repeat(v, EMBED, axis=1)
    repmat = jnp.repeat(jnp.eye(EMBED, dtype=jnp.float32), EMBED, axis=1)

    xs0 = _gather_fn(64)(node_attr.astype(jnp.bfloat16),
                         src2d).reshape(E_PAD, 64)
    msg0, ss1 = _msg0_fn()(ea_pad, xs0, rep(w1t0), w2q0, b2r0, repmat,
                           w1t0, r2(b1_0), r2(gamma_0), r2(beta_0),
                           w1t1, r2(b1_1), r2(gamma_1), r2(beta_1))
    agg0 = _scatter_fn()(msg0, dst2d, zeros_blk)
    x1, x1b = _node_fn(64)(agg0, node_attr, root_0, r2(bias_0))

    w1ts1 = rep(w1t1) * rep(ss1[0:1])
    xs1 = _gather_fn(EMBED)(x1b, src2d).reshape(E_PAD, EMBED)
    msg1 = _msg_fn(EMBED)(ea_pad, xs1, w1ts1, w2q1, b2r1, rep(ss1[1:2]))
    agg1 = _scatter_fn()(msg1, dst2d, zeros_blk)
    return _node_fn(EMBED)(agg1, x1, root_1, r2(bias_1))[0]


# speedup vs baseline: 1.0484x; 1.0365x over previous
import functools

import jax
import jax.numpy as jnp
from jax import lax
from jax.experimental import pallas as pl
from jax.experimental.pallas import tpu as pltpu
from jax.experimental.pallas import tpu_sc as plsc

N_NODES = 10000
N_EDGES = 80000
EDGE_DIM = 16
EMBED = 32
EPS = 1e-5

NUM_CORES = 2
NUM_SUB = 16
NW = NUM_CORES * NUM_SUB
E_PAD = 81920
PER_W = E_PAD // NW
BATCH = 128
NB = PER_W // BATCH
N_PAD = 10240
ROWS_PER_SUB = N_PAD // NUM_SUB
T_MSG = 4096
T_NODE = 2000


def _sc_mesh():
    return plsc.VectorSubcoreMesh(
        core_axis_name="c", subcore_axis_name="s",
        num_cores=NUM_CORES, num_subcores=NUM_SUB)


NSLOT = 4


@functools.cache
def _gather_fn(in_ch):
    def body(x_hbm, idx_hbm, out_hbm, idx_v, rows_v, *sems):
        gsems, wsems = sems[:NSLOT], sems[NSLOT:]
        c = lax.axis_index("c")
        s = lax.axis_index("s")
        w = c * NUM_SUB + s
        pltpu.sync_copy(idx_hbm.at[w], idx_v)

        def g_start(j):
            return pltpu.async_copy(
                x_hbm.at[idx_v.at[j]], rows_v.at[j % NSLOT], gsems[j % NSLOT])

        def w_start(j):
            return pltpu.async_copy(
                rows_v.at[j % NSLOT],
                out_hbm.at[pl.ds(w * PER_W + j * BATCH, BATCH)],
                wsems[j % NSLOT])

        DEPTH = 3
        gd = [None] * NB
        wd = [None] * NB
        for j in range(NB):
            if j >= NSLOT:
                wd[j - NSLOT].wait()
            gd[j] = g_start(j)
            k = j - DEPTH
            if k >= 0:
                gd[k].wait()
                wd[k] = w_start(k)
        for k in range(NB - DEPTH, NB):
            gd[k].wait()
            wd[k] = w_start(k)
        for k in range(NB - NSLOT, NB):
            wd[k].wait()

    return pl.kernel(
        body,
        out_type=jax.ShapeDtypeStruct((E_PAD, in_ch), jnp.bfloat16),
        mesh=_sc_mesh(),
        compiler_params=pltpu.CompilerParams(use_tc_tiling_on_sc=False),
        scratch_types=[
            pltpu.VMEM((NB, BATCH), jnp.int32),
            pltpu.VMEM((NSLOT, BATCH, in_ch), jnp.bfloat16),
        ] + [pltpu.SemaphoreType.DMA] * (2 * NSLOT))


@functools.cache
def _scatter_fn():
    def body(msg_hbm, idx_hbm, zeros_hbm, out_hbm, shared, idx_v, msg_v, ssem):
        c = lax.axis_index("c")
        s = lax.axis_index("s")
        w = c * NUM_SUB + s
        pltpu.sync_copy(zeros_hbm, shared.at[pl.ds(s * ROWS_PER_SUB, ROWS_PER_SUB)])
        plsc.subcore_barrier()
        pltpu.sync_copy(msg_hbm.at[pl.ds(w * PER_W, PER_W)], msg_v)
        pltpu.sync_copy(idx_hbm.at[w], idx_v)

        def step(j, carry):
            pltpu.async_copy(msg_v.at[pl.ds(j * BATCH, BATCH)],
                             shared.at[idx_v.at[j]], ssem, add=True).wait()
            return carry

        lax.fori_loop(0, NB, step, 0)
        plsc.subcore_barrier()
        pltpu.sync_copy(shared.at[pl.ds(s * ROWS_PER_SUB, ROWS_PER_SUB)],
                        out_hbm.at[c, pl.ds(s * ROWS_PER_SUB, ROWS_PER_SUB)])

    return pl.kernel(
        body,
        out_type=jax.ShapeDtypeStruct((NUM_CORES, N_PAD, EMBED), jnp.float32),
        mesh=_sc_mesh(),
        compiler_params=pltpu.CompilerParams(use_tc_tiling_on_sc=False),
        scratch_types=[
            pltpu.VMEM_SHARED((N_PAD, EMBED), jnp.float32),
            pltpu.VMEM((NB, BATCH), jnp.int32),
            pltpu.VMEM((PER_W, EMBED), jnp.float32),
            pltpu.SemaphoreType.DMA,
        ])


def _bn_derive(m_acc, s_acc, w1t, b1, g, be):
    mE = m_acc / N_EDGES
    SE = s_acc / N_EDGES
    t = jnp.dot(mE, w1t, preferred_element_type=jnp.float32)
    mean = t + b1
    bq = jnp.dot(SE, w1t, preferred_element_type=jnp.float32)
    q = jnp.sum(w1t * bq, axis=0, keepdims=True)
    var = (q + 2.0 * b1 * t + b1 * b1) - mean * mean
    scale = g * lax.rsqrt(var + EPS)
    shift = be + scale * (b1 - mean)
    return scale, shift


@functools.cache
def _msg0_fn():
    in_ch = 64
    NT = E_PAD // T_MSG
    grid = (2 * NT,)

    def body(ea_ref, xs_ref, w1tr_ref, w2q_ref, b2r_ref, repmat_ref,
             w1t0_ref, b10_ref, g0_ref, be0_ref,
             w1t1_ref, b11_ref, g1_ref, be1_ref,
             msg_ref, ss1_ref, m_acc, s_acc, screp, shrep):
        i = pl.program_id(0)

        @pl.when(i == 0)
        def _():
            m_acc[...] = jnp.zeros_like(m_acc)
            s_acc[...] = jnp.zeros_like(s_acc)

        @pl.when(i < NT)
        def _():
            ea_t = ea_ref[...]
            m_acc[...] += jnp.sum(ea_t, axis=0, keepdims=True)
            s_acc[...] += lax.dot_general(ea_t, ea_t, (((0,), (0,)), ((), ())),
                                          preferred_element_type=jnp.float32)

        @pl.when(i == NT - 1)
        def _():
            sc0, sh0 = _bn_derive(m_acc[...], s_acc[...], w1t0_ref[...],
                                  b10_ref[...], g0_ref[...], be0_ref[...])
            screp[...] = jnp.dot(sc0, repmat_ref[...],
                                 preferred_element_type=jnp.float32)
            shrep[...] = jnp.dot(sh0, repmat_ref[...],
                                 preferred_element_type=jnp.float32)
            sc1, sh1 = _bn_derive(m_acc[...], s_acc[...], w1t1_ref[...],
                                  b11_ref[...], g1_ref[...], be1_ref[...])
            ss1_ref[...] = jnp.concatenate([sc1, sh1], axis=0)

        @pl.when(i >= NT)
        def _():
            ea = ea_ref[...]
            xs = xs_ref[...]
            CH = 256
            acc = jnp.dot(xs, b2r_ref[...],
                          preferred_element_type=jnp.float32)
            for c in range(0, EMBED * EMBED, CH):
                h0 = jnp.dot(ea, w1tr_ref[:, c:c + CH],
                             preferred_element_type=jnp.float32)
                h_rep = jnp.maximum(
                    h0 * screp[:, c:c + CH] + shrep[:, c:c + CH], 0.0)
                t2 = jnp.dot(xs, w2q_ref[:, c:c + CH],
                             preferred_element_type=jnp.float32)
                p = h_rep * t2
                width = CH
                while width > EMBED:
                    width //= 2
                    p = p[:, :width] + p[:, width:2 * width]
                acc = acc + p
            msg_ref[...] = acc

    small = lambda shape: pl.BlockSpec(shape, lambda i: (0,) * len(shape))
    edge = lambda width: pl.BlockSpec((T_MSG, width),
                                      lambda i: (jnp.maximum(i - NT, 0), 0))
    return pl.pallas_call(
        body,
        grid=grid,
        in_specs=[pl.BlockSpec((T_MSG, EDGE_DIM), lambda i: (i % NT, 0)),
                  edge(in_ch),
                  small((EDGE_DIM, EMBED * EMBED)),
                  small((in_ch, EMBED * EMBED)),
                  small((in_ch, EMBED)),
                  small((EMBED, EMBED * EMBED)),
                  small((EDGE_DIM, EMBED)), small((1, EMBED)),
                  small((1, EMBED)), small((1, EMBED)),
                  small((EDGE_DIM, EMBED)), small((1, EMBED)),
                  small((1, EMBED)), small((1, EMBED))],
        out_specs=[edge(EMBED), small((2, EMBED))],
        out_shape=[jax.ShapeDtypeStruct((E_PAD, EMBED), jnp.float32),
                   jax.ShapeDtypeStruct((2, EMBED), jnp.float32)],
        scratch_shapes=[pltpu.VMEM((1, EDGE_DIM), jnp.float32),
                        pltpu.VMEM((EDGE_DIM, EDGE_DIM), jnp.float32),
                        pltpu.VMEM((1, EMBED * EMBED), jnp.float32),
                        pltpu.VMEM((1, EMBED * EMBED), jnp.float32)],
        compiler_params=pltpu.CompilerParams(
            dimension_semantics=("arbitrary",)),
    )


@functools.cache
def _msg_fn(in_ch):
    grid = (E_PAD // T_MSG,)

    def body(ea_ref, xs_ref, w1ts_ref, w2q_ref, b2r_ref, sh_ref, msg_ref):
        ea = ea_ref[...]
        xs = xs_ref[...]
        CH = 256
        acc = jnp.dot(xs, b2r_ref[...], preferred_element_type=jnp.float32)
        for c in range(0, EMBED * EMBED, CH):
            h0 = jnp.dot(ea, w1ts_ref[:, c:c + CH],
                         preferred_element_type=jnp.float32)
            h_rep = jnp.maximum(h0 + sh_ref[:, c:c + CH], 0.0)
            t2 = jnp.dot(xs, w2q_ref[:, c:c + CH],
                         preferred_element_type=jnp.float32)
            p = h_rep * t2
            width = CH
            while width > EMBED:
                width //= 2
                p = p[:, :width] + p[:, width:2 * width]
            acc = acc + p
        msg_ref[...] = acc

    small = lambda shape: pl.BlockSpec(shape, lambda i: (0, 0))
    return pl.pallas_call(
        body,
        grid=grid,
        in_specs=[pl.BlockSpec((T_MSG, EDGE_DIM), lambda i: (i, 0)),
                  pl.BlockSpec((T_MSG, in_ch), lambda i: (i, 0)),
                  small((EDGE_DIM, EMBED * EMBED)),
                  small((in_ch, EMBED * EMBED)),
                  small((in_ch, EMBED)),
                  small((1, EMBED * EMBED))],
        out_specs=pl.BlockSpec((T_MSG, EMBED), lambda i: (i, 0)),
        out_shape=jax.ShapeDtypeStruct((E_PAD, EMBED), jnp.float32),
        compiler_params=pltpu.CompilerParams(
            dimension_semantics=("arbitrary",)),
    )


@functools.cache
def _node_fn(in_ch):
    grid = (N_NODES // T_NODE,)

    def body(agg_ref, x_ref, root_ref, bias_ref, out_ref, outb_ref):
        a = agg_ref[0] + agg_ref[1]
        r = jnp.dot(x_ref[...], root_ref[...], preferred_element_type=jnp.float32)
        res = jnp.maximum(a + r + bias_ref[...], 0.0)
        out_ref[...] = res
        outb_ref[...] = res.astype(jnp.bfloat16)

    small = lambda shape: pl.BlockSpec(shape, lambda *_: (0,) * len(shape))
    return pl.pallas_call(
        body,
        grid=grid,
        in_specs=[pl.BlockSpec((NUM_CORES, T_NODE, EMBED), lambda i: (0, i, 0)),
                  pl.BlockSpec((T_NODE, in_ch), lambda i: (i, 0)),
                  small((in_ch, EMBED)),
                  small((1, EMBED))],
        out_specs=[pl.BlockSpec((T_NODE, EMBED), lambda i: (i, 0)),
                   pl.BlockSpec((T_NODE, EMBED), lambda i: (i, 0))],
        out_shape=[jax.ShapeDtypeStruct((N_NODES, EMBED), jnp.float32),
                   jax.ShapeDtypeStruct((N_NODES, EMBED), jnp.bfloat16)],
        compiler_params=pltpu.CompilerParams(
            dimension_semantics=("arbitrary",)),
    )


def kernel(node_attr, edge_index, edge_attr,
           w1_0, b1_0, gamma_0, beta_0, w2_0, b2_0, root_0, bias_0,
           w1_1, b1_1, gamma_1, beta_1, w2_1, b2_1, root_1, bias_1):
    src = edge_index[0].astype(jnp.int32)
    dst = edge_index[1].astype(jnp.int32)
    src2d = jnp.pad(src, (0, E_PAD - N_EDGES)).reshape(NW, NB, BATCH)
    dst2d = jnp.pad(dst, (0, E_PAD - N_EDGES),
                    constant_values=N_NODES).reshape(NW, NB, BATCH)
    ea_pad = jnp.pad(edge_attr, ((0, E_PAD - N_EDGES), (0, 0)))
    zeros_blk = jnp.zeros((ROWS_PER_SUB, EMBED), jnp.float32)

    r2 = lambda v: v.reshape(1, EMBED)

    def prep(w2, b2, in_ch):
        w2q = (w2.reshape(in_ch, EMBED, EMBED)
               .transpose(0, 2, 1).reshape(in_ch, EMBED * EMBED))
        return (w2q.astype(jnp.bfloat16),
                b2.reshape(in_ch, EMBED).astype(jnp.bfloat16))

    w1t0, w1t1 = w1_0.T, w1_1.T
    w2q0, b2r0 = prep(w2_0, b2_0, 64)
    w2q1, b2r1 = prep(w2_1, b2_1, EMBED)

    rep = lambda v: jnp.repeat(v, EMBED, axis=1)
    repmat = jnp.repeat(jnp.eye(EMBED, dtype=jnp.float32), EMBED, axis=1)

    xs0 = _gather_fn(64)(node_attr.astype(jnp.bfloat16),
                         src2d).reshape(E_PAD, 64)
    msg0, ss1 = _msg0_fn()(ea_pad, xs0, rep(w1t0), w2q0, b2r0, repmat,
                           w1t0, r2(b1_0), r2(gamma_0), r2(beta_0),
                           w1t1, r2(b1_1), r2(gamma_1), r2(beta_1))
    agg0 = _scatter_fn()(msg0, dst2d, zeros_blk)
    x1, x1b = _node_fn(64)(agg0, node_attr, root_0, r2(bias_0))

    w1ts1 = rep(w1t1) * rep(ss1[0:1])
    xs1 = _gather_fn(EMBED)(x1b, src2d).reshape(E_PAD, EMBED)
    msg1 = _msg_fn(EMBED)(ea_pad, xs1, w1ts1, w2q1, b2r1, rep(ss1[1:2]))
    agg1 = _scatter_fn()(msg1, dst2d, zeros_blk)
    return _node_fn(EMBED)(agg1, x1, root_1, r2(bias_1))[0]
